# Initial kernel scaffold; baseline (speedup 1.0000x reference)
import jax
import jax.numpy as jnp
from jax.experimental import pallas as pl


def kernel(coords, feats, batch_offsets, W_enc, b_enc, W, b):
    raise NotImplementedError("write your pallas kernel here")


# speedup vs baseline: 4.7684x; 4.7684x over previous
import functools

import jax
import jax.numpy as jnp
from jax.experimental import pallas as pl
from jax.experimental.pallas import tpu as pltpu

N = 32768
B = 16
D_IN = 64
D_COORD = 3
D_EMB = 128
NUM_CLASSES = 55

BLK = 4096
GRID = N // BLK


def _fused_kernel(off_ref, coords_ref, feats_ref, wenc_ref, benc_ref,
                  w_ref, b_ref, out_ref, acc_ref):
    g = pl.program_id(0)

    @pl.when(g == 0)
    def _init():
        acc_ref[...] = jnp.zeros_like(acc_ref)

    wc = wenc_ref[0:D_COORD, :]
    wf = wenc_ref[D_COORD:D_COORD + D_IN, :]
    emb = jnp.dot(coords_ref[...], wc, preferred_element_type=jnp.float32)
    emb += jnp.dot(feats_ref[...], wf, preferred_element_type=jnp.float32)
    emb = jnp.maximum(emb + benc_ref[...], 0.0)

    bidx = jax.lax.broadcasted_iota(jnp.int32, (B, 1), 0)
    lower = jnp.zeros((B, 1), jnp.int32)
    upper = jnp.zeros((B, 1), jnp.int32)
    for k in range(B):
        lower = jnp.where(bidx == k, off_ref[k], lower)
        upper = jnp.where(bidx == k, off_ref[k + 1], upper)

    rows = g * BLK + jax.lax.broadcasted_iota(jnp.int32, (1, BLK), 1)
    sel = ((rows >= lower) & (rows < upper)).astype(jnp.float32)

    acc_ref[...] += jax.lax.dot_general(
        sel, emb, (((1,), (0,)), ((), ())),
        preferred_element_type=jnp.float32)

    @pl.when(g == GRID - 1)
    def _finish():
        counts = (upper - lower).astype(jnp.float32)
        gf = acc_ref[...] / jnp.maximum(counts, 1.0)
        out_ref[...] = jnp.dot(gf, w_ref[...],
                               preferred_element_type=jnp.float32) + b_ref[...]


@jax.jit
def kernel(coords, feats, batch_offsets, W_enc, b_enc, W, b):
    return pl.pallas_call(
        _fused_kernel,
        grid=(GRID,),
        in_specs=[
            pl.BlockSpec(memory_space=pltpu.SMEM),
            pl.BlockSpec((BLK, D_COORD), lambda g: (g, 0)),
            pl.BlockSpec((BLK, D_IN), lambda g: (g, 0)),
            pl.BlockSpec((D_COORD + D_IN, D_EMB), lambda g: (0, 0)),
            pl.BlockSpec((1, D_EMB), lambda g: (0, 0)),
            pl.BlockSpec((D_EMB, NUM_CLASSES), lambda g: (0, 0)),
            pl.BlockSpec((1, NUM_CLASSES), lambda g: (0, 0)),
        ],
        out_specs=pl.BlockSpec((B, NUM_CLASSES), lambda g: (0, 0)),
        out_shape=jax.ShapeDtypeStruct((B, NUM_CLASSES), jnp.float32),
        scratch_shapes=[pltpu.VMEM((B, D_EMB), jnp.float32)],
        compiler_params=pltpu.CompilerParams(
            dimension_semantics=("arbitrary",)),
    )(batch_offsets, coords, feats, W_enc,
      b_enc.reshape(1, D_EMB), W, b.reshape(1, NUM_CLASSES))


# speedup vs baseline: 6.8991x; 1.4468x over previous
import functools

import jax
import jax.numpy as jnp
from jax.experimental import pallas as pl
from jax.experimental.pallas import tpu as pltpu

N = 32768
B = 16
D_IN = 64
D_COORD = 3
D_EMB = 128
NUM_CLASSES = 55

BLK = 4096
GRID = N // BLK


def _fused_kernel(off_ref, coords_ref, feats_ref, wenc_ref, benc_ref,
                  w_ref, b_ref, out_ref, acc_ref):
    g = pl.program_id(0)

    @pl.when(g == 0)
    def _init():
        acc_ref[...] = jnp.zeros_like(acc_ref)

    wc = wenc_ref[0:D_COORD, :]
    wf = wenc_ref[D_COORD:D_COORD + D_IN, :]
    emb = jax.lax.dot_general(coords_ref[...], wc, (((0,), (0,)), ((), ())),
                              preferred_element_type=jnp.float32)
    emb += jnp.dot(feats_ref[...], wf, preferred_element_type=jnp.float32)
    emb = jnp.maximum(emb + benc_ref[...], 0.0)

    bidx = jax.lax.broadcasted_iota(jnp.int32, (B, 1), 0)
    lower = jnp.zeros((B, 1), jnp.int32)
    upper = jnp.zeros((B, 1), jnp.int32)
    for k in range(B):
        lower = jnp.where(bidx == k, off_ref[k], lower)
        upper = jnp.where(bidx == k, off_ref[k + 1], upper)

    rows = g * BLK + jax.lax.broadcasted_iota(jnp.int32, (1, BLK), 1)
    sel = ((rows >= lower) & (rows < upper)).astype(jnp.float32)

    acc_ref[...] += jax.lax.dot_general(
        sel, emb, (((1,), (0,)), ((), ())),
        preferred_element_type=jnp.float32)

    @pl.when(g == GRID - 1)
    def _finish():
        counts = (upper - lower).astype(jnp.float32)
        gf = acc_ref[...] / jnp.maximum(counts, 1.0)
        out_ref[...] = jnp.dot(gf, w_ref[...],
                               preferred_element_type=jnp.float32) + b_ref[...]


@jax.jit
def kernel(coords, feats, batch_offsets, W_enc, b_enc, W, b):
    return pl.pallas_call(
        _fused_kernel,
        grid=(GRID,),
        in_specs=[
            pl.BlockSpec(memory_space=pltpu.SMEM),
            pl.BlockSpec((D_COORD, BLK), lambda g: (0, g)),
            pl.BlockSpec((BLK, D_IN), lambda g: (g, 0)),
            pl.BlockSpec((D_COORD + D_IN, D_EMB), lambda g: (0, 0)),
            pl.BlockSpec((1, D_EMB), lambda g: (0, 0)),
            pl.BlockSpec((D_EMB, NUM_CLASSES), lambda g: (0, 0)),
            pl.BlockSpec((1, NUM_CLASSES), lambda g: (0, 0)),
        ],
        out_specs=pl.BlockSpec((B, NUM_CLASSES), lambda g: (0, 0)),
        out_shape=jax.ShapeDtypeStruct((B, NUM_CLASSES), jnp.float32),
        scratch_shapes=[pltpu.VMEM((B, D_EMB), jnp.float32)],
        compiler_params=pltpu.CompilerParams(
            dimension_semantics=("arbitrary",)),
    )(batch_offsets, coords.T, feats, W_enc,
      b_enc.reshape(1, D_EMB), W, b.reshape(1, NUM_CLASSES))


# speedup vs baseline: 7.3209x; 1.0611x over previous
import functools

import jax
import jax.numpy as jnp
from jax.experimental import pallas as pl
from jax.experimental.pallas import tpu as pltpu

N = 32768
B = 16
D_IN = 64
D_COORD = 3
D_EMB = 128
NUM_CLASSES = 55

BLK = 8192
GRID = N // BLK


def _fused_kernel(off_ref, coords_ref, feats_ref, wenc_ref, benc_ref,
                  w_ref, b_ref, out_ref, acc_ref):
    g = pl.program_id(0)

    @pl.when(g == 0)
    def _init():
        acc_ref[...] = jnp.zeros_like(acc_ref)

    wc = wenc_ref[0:D_COORD, :]
    wf = wenc_ref[D_COORD:D_COORD + D_IN, :]
    emb = jax.lax.dot_general(coords_ref[...], wc, (((0,), (0,)), ((), ())),
                              preferred_element_type=jnp.float32)
    emb += jnp.dot(feats_ref[...], wf, preferred_element_type=jnp.float32)
    emb = jnp.maximum(emb + benc_ref[...], 0.0)

    bidx = jax.lax.broadcasted_iota(jnp.int32, (B, 1), 0)
    lower = jnp.zeros((B, 1), jnp.int32)
    upper = jnp.zeros((B, 1), jnp.int32)
    for k in range(B):
        lower = jnp.where(bidx == k, off_ref[k], lower)
        upper = jnp.where(bidx == k, off_ref[k + 1], upper)

    rows = g * BLK + jax.lax.broadcasted_iota(jnp.int32, (1, BLK), 1)
    sel = ((rows >= lower) & (rows < upper)).astype(jnp.float32)

    acc_ref[...] += jax.lax.dot_general(
        sel, emb, (((1,), (0,)), ((), ())),
        preferred_element_type=jnp.float32)

    @pl.when(g == GRID - 1)
    def _finish():
        counts = (upper - lower).astype(jnp.float32)
        gf = acc_ref[...] / jnp.maximum(counts, 1.0)
        out_ref[...] = jnp.dot(gf, w_ref[...],
                               preferred_element_type=jnp.float32) + b_ref[...]


@jax.jit
def kernel(coords, feats, batch_offsets, W_enc, b_enc, W, b):
    return pl.pallas_call(
        _fused_kernel,
        grid=(GRID,),
        in_specs=[
            pl.BlockSpec(memory_space=pltpu.SMEM),
            pl.BlockSpec((D_COORD, BLK), lambda g: (0, g)),
            pl.BlockSpec((BLK, D_IN), lambda g: (g, 0)),
            pl.BlockSpec((D_COORD + D_IN, D_EMB), lambda g: (0, 0)),
            pl.BlockSpec((1, D_EMB), lambda g: (0, 0)),
            pl.BlockSpec((D_EMB, NUM_CLASSES), lambda g: (0, 0)),
            pl.BlockSpec((1, NUM_CLASSES), lambda g: (0, 0)),
        ],
        out_specs=pl.BlockSpec((B, NUM_CLASSES), lambda g: (0, 0)),
        out_shape=jax.ShapeDtypeStruct((B, NUM_CLASSES), jnp.float32),
        scratch_shapes=[pltpu.VMEM((B, D_EMB), jnp.float32)],
        compiler_params=pltpu.CompilerParams(
            dimension_semantics=("arbitrary",)),
    )(batch_offsets, coords.T, feats, W_enc,
      b_enc.reshape(1, D_EMB), W, b.reshape(1, NUM_CLASSES))


# speedup vs baseline: 12.0775x; 1.6497x over previous
import jax
import jax.numpy as jnp
from jax.experimental import pallas as pl
from jax.experimental.pallas import tpu as pltpu

N = 32768
B = 16
D_IN = 64
D_COORD = 3
D_EMB = 128
NUM_CLASSES = 55

BLK = 8192
GRID = N // BLK


def _fused_kernel(off_ref, coords_ref, feats_ref, wenc_ref, benc_ref,
                  w_ref, b_ref, out_ref, acc_ref):
    g = pl.program_id(0)

    @pl.when(g == 0)
    def _init():
        acc_ref[...] = jnp.zeros_like(acc_ref)

    wc = wenc_ref[0:D_COORD, :]
    wf = wenc_ref[D_COORD:D_COORD + D_IN, :]
    emb = jax.lax.dot_general(coords_ref[...], wc, (((0,), (0,)), ((), ())),
                              preferred_element_type=jnp.float32)
    emb += jax.lax.dot_general(feats_ref[...], wf, (((0,), (0,)), ((), ())),
                               preferred_element_type=jnp.float32)
    emb = jnp.maximum(emb + benc_ref[...], 0.0)

    bidx = jax.lax.broadcasted_iota(jnp.int32, (B, 1), 0)
    lower = jnp.zeros((B, 1), jnp.int32)
    upper = jnp.zeros((B, 1), jnp.int32)
    for k in range(B):
        lower = jnp.where(bidx == k, off_ref[k], lower)
        upper = jnp.where(bidx == k, off_ref[k + 1], upper)

    rows = g * BLK + jax.lax.broadcasted_iota(jnp.int32, (1, BLK), 1)
    sel = ((rows >= lower) & (rows < upper)).astype(jnp.float32)

    acc_ref[...] += jax.lax.dot_general(
        sel, emb, (((1,), (0,)), ((), ())),
        preferred_element_type=jnp.float32)

    @pl.when(g == GRID - 1)
    def _finish():
        counts = (upper - lower).astype(jnp.float32)
        gf = acc_ref[...] / jnp.maximum(counts, 1.0)
        out_ref[...] = jnp.dot(gf, w_ref[...],
                               preferred_element_type=jnp.float32) + b_ref[...]


@jax.jit
def kernel(coords, feats, batch_offsets, W_enc, b_enc, W, b):
    return pl.pallas_call(
        _fused_kernel,
        grid=(GRID,),
        in_specs=[
            pl.BlockSpec(memory_space=pltpu.SMEM),
            pl.BlockSpec((D_COORD, BLK), lambda g: (0, g)),
            pl.BlockSpec((D_IN, BLK), lambda g: (0, g)),
            pl.BlockSpec((D_COORD + D_IN, D_EMB), lambda g: (0, 0)),
            pl.BlockSpec((1, D_EMB), lambda g: (0, 0)),
            pl.BlockSpec((D_EMB, NUM_CLASSES), lambda g: (0, 0)),
            pl.BlockSpec((1, NUM_CLASSES), lambda g: (0, 0)),
        ],
        out_specs=pl.BlockSpec((B, NUM_CLASSES), lambda g: (0, 0)),
        out_shape=jax.ShapeDtypeStruct((B, NUM_CLASSES), jnp.float32),
        scratch_shapes=[pltpu.VMEM((B, D_EMB), jnp.float32)],
        compiler_params=pltpu.CompilerParams(
            dimension_semantics=("arbitrary",)),
    )(batch_offsets, coords.T, feats.T, W_enc,
      b_enc.reshape(1, D_EMB), W, b.reshape(1, NUM_CLASSES))
